# baseline (speedup 1.0000x reference)
import jax
import jax.numpy as jnp
from jax.experimental import pallas as pl
from jax.experimental.pallas import tpu as pltpu

B, W = 4, 2048
D_MODEL = 768
DQ = D_MODEL // 4
NROWS = 64
BIGI = 1 << 20


def _kernel(tags_ref, slen_ref, depth_ref, vdistp_ref, conj_ref, relw_ref,
            relb_ref, fusew_ref, fuseb_ref, lng_ref, lnb_ref, out_ref,
            table_ref):
    f32 = jnp.float32

    @pl.when(pl.program_id(0) == 0)
    def _fold():
        wd = fusew_ref[0:DQ, :]
        wv = fusew_ref[DQ:2 * DQ, :]
        wc = fusew_ref[2 * DQ:3 * DQ, :]
        wr = fusew_ref[3 * DQ:4 * DQ, :]
        a_d = jax.lax.dot(depth_ref[...], wd, preferred_element_type=f32)
        a_v = jax.lax.dot(vdistp_ref[...], wv, preferred_element_type=f32)
        a_c = jax.lax.dot(conj_ref[...], wc, preferred_element_type=f32)
        a_r = jax.lax.dot(relw_ref[...], wr, preferred_element_type=f32)
        bias = fuseb_ref[...] + jax.lax.dot(relb_ref[...], wr,
                                            preferred_element_type=f32)
        pad = jnp.zeros((NROWS - 58, D_MODEL), f32)
        table_ref[...] = jnp.concatenate([a_d, a_v, a_c, a_r, bias, pad],
                                         axis=0)

    t = tags_ref[0]
    iota_l = jax.lax.broadcasted_iota(jnp.int32, (1, W), 1)

    def shift_r(x, k, fill):
        return jnp.where(iota_l >= k, jnp.roll(x, k, axis=1), fill)

    def shift_l(x, k, fill):
        return jnp.where(iota_l < (W - k), jnp.roll(x, -k, axis=1), fill)

    def cumsum(x):
        c = x
        k = 1
        while k < W:
            c = c + shift_r(c, k, 0)
            k *= 2
        return c

    didx = jnp.clip(cumsum((t == 15).astype(jnp.int32)), 0, 7)
    cidx = jnp.clip(cumsum((t == 9).astype(jnp.int32)), 0, 7)

    isv = (t == 10) | (t == 11)
    vpos_f = jnp.where(isv, iota_l, -BIGI)
    vpos_b = jnp.where(isv, iota_l, BIGI)
    k = 1
    while k < W:
        vpos_f = jnp.maximum(vpos_f, shift_r(vpos_f, k, -BIGI))
        vpos_b = jnp.minimum(vpos_b, shift_l(vpos_b, k, BIGI))
        k *= 2
    ld = iota_l - vpos_f
    rd = vpos_b - iota_l
    sd = jnp.where(ld <= rd, ld, -rd)
    vd = jnp.where(jnp.any(isv), sd, 0)
    vidx = jnp.clip(vd, -16, 16) + 16

    rp = iota_l.astype(f32) / jnp.maximum(slen_ref[0, 0, 0], 1.0)

    oh_d = (jax.lax.broadcasted_iota(jnp.int32, (8, W), 0) == didx).astype(f32)
    oh_v = (jax.lax.broadcasted_iota(jnp.int32, (40, W), 0) == vidx).astype(f32)
    oh_c = (jax.lax.broadcasted_iota(jnp.int32, (8, W), 0) == cidx).astype(f32)
    oh = jnp.concatenate(
        [oh_d, oh_v, oh_c, rp, jnp.ones((1, W), f32),
         jnp.zeros((NROWS - 58, W), f32)], axis=0)

    h = jax.lax.dot_general(oh, table_ref[...], (((0,), (0,)), ((), ())),
                            preferred_element_type=f32)
    g = 0.5 * h * (1.0 + jax.lax.erf(h * 0.7071067811865476))
    mu = jnp.mean(g, axis=1, keepdims=True)
    d = g - mu
    var = jnp.mean(d * d, axis=1, keepdims=True)
    out_ref[0] = d * jax.lax.rsqrt(var + 1e-5) * lng_ref[...] + lnb_ref[...]


@jax.jit
def kernel(word_ids, pos_tags, seq_lengths, mask, depth_table, vdist_table,
           conj_table, rel_W, rel_b, fuse_W, fuse_b, ln_g, ln_b):
    f32 = jnp.float32
    vdist_p = jnp.pad(vdist_table, ((0, 40 - 33), (0, 0)))
    tags3 = pos_tags.astype(jnp.int32).reshape(B, 1, W)
    slen3 = seq_lengths.astype(f32).reshape(B, 1, 1)

    const = lambda shape: pl.BlockSpec(shape, lambda b: tuple(0 for _ in shape))
    out = pl.pallas_call(
        _kernel,
        grid=(B,),
        in_specs=[
            pl.BlockSpec((1, 1, W), lambda b: (b, 0, 0)),
            pl.BlockSpec((1, 1, 1), lambda b: (b, 0, 0)),
            const((8, DQ)),
            const((40, DQ)),
            const((8, DQ)),
            const((1, DQ)),
            const((1, DQ)),
            const((D_MODEL, D_MODEL)),
            const((1, D_MODEL)),
            const((1, D_MODEL)),
            const((1, D_MODEL)),
        ],
        out_specs=pl.BlockSpec((1, W, D_MODEL), lambda b: (b, 0, 0)),
        out_shape=jax.ShapeDtypeStruct((B, W, D_MODEL), f32),
        scratch_shapes=[pltpu.VMEM((NROWS, D_MODEL), f32)],
    )(tags3, slen3, depth_table, vdist_p, conj_table, rel_W,
      rel_b.reshape(1, DQ), fuse_W, fuse_b.reshape(1, D_MODEL),
      ln_g.reshape(1, D_MODEL), ln_b.reshape(1, D_MODEL))
    return out
